# Initial kernel scaffold; baseline (speedup 1.0000x reference)
import functools

import jax
import jax.numpy as jnp
from jax import lax
from jax.experimental import pallas as pl
from jax.experimental.pallas import tpu as pltpu


def _heat_body(x_ref, heat_ref):
    heat_ref[0] = jnp.sum(x_ref[0], axis=0)


def _select_body(heat_ref, scale_ref, *, k, hw_bits, W):
    h = heat_ref[0]
    key = jnp.where(h > 0, lax.bitcast_convert_type(h, jnp.int32), 0)

    def t_step(i, t):
        cand = t | (jnp.int32(1) << (jnp.int32(30) - i))
        cnt = jnp.sum((key >= cand).astype(jnp.int32))
        return jnp.where(cnt >= k, cand, t)

    T = lax.fori_loop(0, 31, t_step, jnp.int32(0))

    count_gt = jnp.sum((key > T).astype(jnp.int32))
    need = k - count_gt
    eq = key == T
    idx = (lax.broadcasted_iota(jnp.int32, h.shape, 0) * W
           + lax.broadcasted_iota(jnp.int32, h.shape, 1))

    def m_step(i, m):
        cand = m | (jnp.int32(1) << (jnp.int32(hw_bits - 1) - i))
        cnt = jnp.sum((eq & (idx < cand)).astype(jnp.int32))
        return jnp.where(cnt < need, cand, m)

    m = lax.fori_loop(0, hw_bits, m_step, jnp.int32(0))
    selected = (key > T) | (eq & (idx <= m))
    scale_ref[0] = jnp.where(selected, jnp.float32(2.0), jnp.float32(1.0))


def _scale_body(x_ref, scale_ref, out_ref):
    out_ref[0] = x_ref[0] * scale_ref[0][None, :, :]


@jax.jit
def kernel(x):
    B, C, H, W = x.shape
    k = int(H * W * 0.15)
    hw_bits = max((H * W - 1).bit_length(), 1)
    bh = 48 if H % 48 == 0 else H
    nh = H // bh

    heat = pl.pallas_call(
        _heat_body,
        grid=(B, nh),
        in_specs=[pl.BlockSpec((1, C, bh, W), lambda b, i: (b, 0, i, 0))],
        out_specs=pl.BlockSpec((1, bh, W), lambda b, i: (b, i, 0)),
        out_shape=jax.ShapeDtypeStruct((B, H, W), jnp.float32),
        compiler_params=pltpu.CompilerParams(
            dimension_semantics=("parallel", "parallel")),
    )(x)

    scale = pl.pallas_call(
        functools.partial(_select_body, k=k, hw_bits=hw_bits, W=W),
        grid=(B,),
        in_specs=[pl.BlockSpec((1, H, W), lambda b: (b, 0, 0))],
        out_specs=pl.BlockSpec((1, H, W), lambda b: (b, 0, 0)),
        out_shape=jax.ShapeDtypeStruct((B, H, W), jnp.float32),
        compiler_params=pltpu.CompilerParams(
            dimension_semantics=("parallel",)),
    )(heat)

    out = pl.pallas_call(
        _scale_body,
        grid=(B, nh),
        in_specs=[
            pl.BlockSpec((1, C, bh, W), lambda b, i: (b, 0, i, 0)),
            pl.BlockSpec((1, bh, W), lambda b, i: (b, i, 0)),
        ],
        out_specs=pl.BlockSpec((1, C, bh, W), lambda b, i: (b, 0, i, 0)),
        out_shape=jax.ShapeDtypeStruct((B, C, H, W), x.dtype),
        compiler_params=pltpu.CompilerParams(
            dimension_semantics=("parallel", "parallel")),
    )(x)

    return out


# speedup vs baseline: 27.0856x; 27.0856x over previous
import functools

import jax
import jax.numpy as jnp
from jax import lax
from jax.experimental import pallas as pl
from jax.experimental.pallas import tpu as pltpu


def _heat_body(x_ref, heat_ref):
    heat_ref[0] = jnp.sum(x_ref[0], axis=0)


def _select_body(heat_ref, scale_ref, *, k, hw_bits, W):
    h = heat_ref[0]
    key = jnp.where(h > 0, lax.bitcast_convert_type(h, jnp.int32), 0)

    def t_step(i, t):
        cand = t | (jnp.int32(1) << (jnp.int32(30) - i))
        cnt = jnp.sum((key >= cand).astype(jnp.int32))
        return jnp.where(cnt >= k, cand, t)

    T = lax.fori_loop(0, 31, t_step, jnp.int32(0))

    count_gt = jnp.sum((key > T).astype(jnp.int32))
    need = k - count_gt
    eq = key == T
    idx = (lax.broadcasted_iota(jnp.int32, h.shape, 0) * W
           + lax.broadcasted_iota(jnp.int32, h.shape, 1))

    def m_step(i, m):
        cand = m | (jnp.int32(1) << (jnp.int32(hw_bits - 1) - i))
        cnt = jnp.sum((eq & (idx < cand)).astype(jnp.int32))
        return jnp.where(cnt < need, cand, m)

    m = lax.fori_loop(0, hw_bits, m_step, jnp.int32(0))
    selected = (key > T) | (eq & (idx <= m))
    scale_ref[0] = jnp.where(selected, jnp.float32(2.0), jnp.float32(1.0))


def _scale_body(x_ref, scale_ref, out_ref):
    out_ref[0] = x_ref[0] * scale_ref[0][None, :, :]


@jax.jit
def kernel(x):
    B, C, H, W = x.shape
    k = int(H * W * 0.15)
    hw_bits = max((H * W - 1).bit_length(), 1)
    bh = 48 if H % 48 == 0 else H
    nh = H // bh

    heat = pl.pallas_call(
        _heat_body,
        grid=(B, nh),
        in_specs=[pl.BlockSpec((1, C, bh, W), lambda b, i: (b, 0, i, 0))],
        out_specs=pl.BlockSpec((1, bh, W), lambda b, i: (b, i, 0)),
        out_shape=jax.ShapeDtypeStruct((B, H, W), jnp.float32),
        compiler_params=pltpu.CompilerParams(
            dimension_semantics=("parallel", "parallel")),
    )(x)

    scale = pl.pallas_call(
        functools.partial(_select_body, k=k, hw_bits=hw_bits, W=W),
        grid=(B,),
        in_specs=[pl.BlockSpec((1, H, W), lambda b: (b, 0, 0))],
        out_specs=pl.BlockSpec((1, H, W), lambda b: (b, 0, 0)),
        out_shape=jax.ShapeDtypeStruct((B, H, W), jnp.float32),
        compiler_params=pltpu.CompilerParams(
            dimension_semantics=("parallel",)),
    )(heat)

    out = pl.pallas_call(
        _scale_body,
        grid=(B, nh),
        in_specs=[
            pl.BlockSpec((1, C, bh, W), lambda b, i: (b, 0, i, 0)),
            pl.BlockSpec((1, bh, W), lambda b, i: (b, i, 0)),
        ],
        out_specs=pl.BlockSpec((1, C, bh, W), lambda b, i: (b, 0, i, 0)),
        out_shape=jax.ShapeDtypeStruct((B, C, H, W), x.dtype),
        compiler_params=pltpu.CompilerParams(
            dimension_semantics=("parallel", "parallel")),
    )(x, scale)

    return out
